# baseline (speedup 1.0000x reference)
import functools

import numpy as np
import jax
import jax.numpy as jnp
from jax import lax
from jax.experimental import pallas as pl
from jax.experimental.pallas import tpu as pltpu
from jax.experimental.pallas import tpu_sc as plsc

B = 1024
D_IN = 2048
D_H = 512
HALF = B // 2
THR = np.float32(0.2)
L = 16
RB = 128
NRB = B // RB

_PERM = np.random.RandomState(0).permutation(B).astype(np.int32)

_BOOT = []
_mem = {0: [], 1: []}
for _p in range(B):
    if len(_mem[0]) >= 2 and len(_mem[1]) >= 2:
        break
    _idx = int(_PERM[_p])
    _c = 0 if _idx < HALF else 1
    if len(_mem[_c]) >= 2:
        raise AssertionError("bootstrap interrupted; generic prologue needed")
    _mem[_c].append(_idx)
    _BOOT.append((_idx, _c, [list(_mem[0]), list(_mem[1])]))
_Q = len(_BOOT)
_A_ROWS = _mem[0]
_N_ROWS = _mem[1]
_SEED_ROWS = _A_ROWS + _N_ROWS


def _boot_products(sim_rows_of):
    binit_a = jnp.maximum(sim_rows_of(_A_ROWS[0]), sim_rows_of(_A_ROWS[1]))
    binit_n = jnp.maximum(sim_rows_of(_N_ROWS[0]), sim_rows_of(_N_ROWS[1]))
    pos = lax.broadcasted_iota(jnp.int32, (1, B), 1)
    oinit = jnp.zeros((1, B), jnp.float32)
    for _idx, _c, (_ma, _mn) in _BOOT:
        a_s = (jnp.float32(0.5) if not _ma
               else functools.reduce(jnp.maximum,
                                     [sim_rows_of(k)[0, _idx] for k in _ma]))
        n_s = (jnp.float32(0.5) if not _mn
               else functools.reduce(jnp.maximum,
                                     [sim_rows_of(k)[0, _idx] for k in _mn]))
        oinit = jnp.where(pos == _idx, a_s - n_s, oinit)
    return binit_a, binit_n, oinit


def _h_body(x_ref, w_ref, b_ref, binit_ref, oinit_ref, hfull, n2full):
    i = pl.program_id(0)
    h = lax.dot_general(x_ref[...], w_ref[...], (((1,), (1,)), ((), ())),
                        preferred_element_type=jnp.float32)
    h = jnp.maximum(h + b_ref[...], 0.0)
    hfull[pl.ds(i * RB, RB), :] = h
    n2full[pl.ds(i * RB, RB), :] = jnp.sum(h * h, axis=1, keepdims=True)

    @pl.when(i == NRB - 1)
    def _tail():
        hf = hfull[...]
        norms = jnp.sqrt(n2full[...])

        ks = sorted(set(_SEED_ROWS))
        hks = jnp.concatenate([hfull[pl.ds(k, 1), :] for k in ks], axis=0)
        nums = lax.dot_general(hks, hf, (((1,), (1,)), ((), ())),
                               preferred_element_type=jnp.float32)
        nks = jnp.concatenate([norms[k:k + 1, :] for k in ks], axis=0)
        sims = nums / jnp.maximum(nks * norms.T, 1e-8)
        rows = {k: sims[r:r + 1, :] for r, k in enumerate(ks)}
        binit_a, binit_n, oinit = _boot_products(lambda k: rows[k])
        binit_ref[0:1, :] = binit_a
        binit_ref[1:2, :] = binit_n
        oinit_ref[...] = oinit


def _h_and_boot(x, W0, b0):
    return pl.pallas_call(
        _h_body,
        grid=(NRB,),
        in_specs=[
            pl.BlockSpec((RB, D_IN), lambda i: (i, 0)),
            pl.BlockSpec((D_H, D_IN), lambda i: (0, 0)),
            pl.BlockSpec((1, D_H), lambda i: (0, 0)),
        ],
        out_specs=[
            pl.BlockSpec((2, B), lambda i: (0, 0)),
            pl.BlockSpec((1, B), lambda i: (0, 0)),
        ],
        out_shape=(
            jax.ShapeDtypeStruct((2, B), jnp.float32),
            jax.ShapeDtypeStruct((1, B), jnp.float32),
        ),
        scratch_shapes=[pltpu.VMEM((B, D_H), jnp.float32),
                        pltpu.VMEM((B, 1), jnp.float32)],
    )(x, W0, b0.reshape(1, D_H))


def _s_body(x_ref, w_ref, b_ref, s_ref):
    hf = lax.dot_general(x_ref[...], w_ref[...], (((1,), (1,)), ((), ())),
                         preferred_element_type=jnp.float32)
    hf = jnp.maximum(hf + b_ref[...], 0.0)
    g = lax.dot_general(hf, hf, (((1,), (1,)), ((), ())),
                        preferred_element_type=jnp.float32)
    norms = jnp.sqrt(jnp.sum(hf * hf, axis=1, keepdims=True))
    den = jnp.maximum(norms * norms.T, 1e-8)
    s_ref[...] = g / den


def _full_sim(x, W0, b0):
    return pl.pallas_call(
        _s_body,
        out_shape=jax.ShapeDtypeStruct((B, B), jnp.float32),
    )(x, W0, b0.reshape(1, D_H))


def _sweep_body(perm_hbm, binit_hbm, oinit_hbm, out_hbm, j_hbm,
                perm_v, best_a, best_n, outs_v, j_v, sem):
    wid = lax.axis_index("s") * 2 + lax.axis_index("c")

    @pl.when(wid == 0)
    def _run():
        c1 = pltpu.async_copy(perm_hbm, perm_v, sem)
        c2 = pltpu.async_copy(binit_hbm.at[0], best_a, sem)
        c3 = pltpu.async_copy(binit_hbm.at[1], best_n, sem)
        c4 = pltpu.async_copy(oinit_hbm.at[0], outs_v, sem)
        c1.wait()
        c2.wait()
        c3.wait()
        c4.wait()

        lane = lax.iota(jnp.int32, L)

        @plsc.parallel_loop(0, B // L, 1, unroll=4,
                            carry=jnp.full((L,), B, jnp.int32))
        def acc(k, acc_v):
            pos = k * L + lane
            idxs = perm_v[pl.ds(k * L, L)]
            isa = idxs < HALF
            ba = plsc.load_gather(best_a, [idxs])
            bn = plsc.load_gather(best_n, [idxs])
            cs = jnp.where(isa, ba, bn)
            live = pos >= _Q
            flag = (cs < THR) & live
            plsc.store_scatter(outs_v, [idxs], ba - bn, mask=live)
            return jnp.minimum(acc_v, jnp.where(flag, pos, B))

        j_v[...] = acc
        pltpu.sync_copy(outs_v, out_hbm)
        pltpu.sync_copy(j_v, j_hbm)


def _fast_sweep(binit, oinit):
    mesh = plsc.VectorSubcoreMesh(core_axis_name="c", subcore_axis_name="s", num_cores=1)
    perm = jnp.asarray(_PERM)
    run = pl.kernel(
        _sweep_body,
        out_type=(jax.ShapeDtypeStruct((B,), jnp.float32),
                  jax.ShapeDtypeStruct((L,), jnp.int32)),
        mesh=mesh,
        scratch_types=[
            pltpu.VMEM((B,), jnp.int32),
            pltpu.VMEM((B,), jnp.float32),
            pltpu.VMEM((B,), jnp.float32),
            pltpu.VMEM((B,), jnp.float32),
            pltpu.VMEM((L,), jnp.int32),
            pltpu.SemaphoreType.DMA,
        ],
        compiler_params=pltpu.CompilerParams(needs_layout_passes=False),
    )
    return run(perm, binit, oinit)


def _loop_body(s_hbm, perm_hbm, binit_hbm, oinit_hbm, out_hbm,
               perm_v, best_a, best_n, outs_v, row_v, sem):
    wid = lax.axis_index("s") * 2 + lax.axis_index("c")

    @pl.when(wid == 0)
    def _run():
        c1 = pltpu.async_copy(perm_hbm, perm_v, sem)
        c2 = pltpu.async_copy(binit_hbm.at[0], best_a, sem)
        c3 = pltpu.async_copy(binit_hbm.at[1], best_n, sem)
        c4 = pltpu.async_copy(oinit_hbm.at[0], outs_v, sem)
        c1.wait()
        c2.wait()
        c3.wait()
        c4.wait()

        lane = lax.iota(jnp.int32, L)
        lane0 = lane == 0

        def pass_body(carry):
            s, count_a, count_n = carry

            def chunk(k, acc):
                pos = k * L + lane
                idxs = perm_v[pl.ds(k * L, L)]
                isa = idxs < HALF
                ba = plsc.load_gather(best_a, [idxs])
                bn = plsc.load_gather(best_n, [idxs])
                cs = jnp.where(isa, ba, bn)
                cnt = jnp.where(isa, count_a, count_n)
                live = pos >= s
                flag = ((cnt <= 1) | (cs < THR)) & live
                plsc.store_scatter(outs_v, [idxs], ba - bn, mask=live)
                return jnp.minimum(acc, jnp.where(flag, pos, B))

            acc = lax.fori_loop(0, B // L, chunk,
                                jnp.full((L,), B, jnp.int32))
            j = jnp.min(acc)
            found = j < B

            def do_append(c):
                count_a, count_n = c
                j16 = jnp.full((L,), 0, jnp.int32) + j
                idx16 = plsc.load_gather(perm_v, [j16])
                idx_s = jnp.max(idx16)
                is_a = idx_s < HALF

                pltpu.sync_copy(s_hbm.at[idx_s], row_v)

                @pl.when(is_a)
                def _ua():
                    for k in range(B // L):
                        sl = pl.ds(k * L, L)
                        best_a[sl] = jnp.maximum(best_a[sl], row_v[sl])

                @pl.when(jnp.logical_not(is_a))
                def _un():
                    for k in range(B // L):
                        sl = pl.ds(k * L, L)
                        best_n[sl] = jnp.maximum(best_n[sl], row_v[sl])

                count_a = count_a + jnp.where(is_a, 1, 0)
                count_n = count_n + jnp.where(is_a, 0, 1)
                a16 = plsc.load_gather(best_a, [idx16])
                n16 = plsc.load_gather(best_n, [idx16])
                plsc.store_scatter(outs_v, [idx16], a16 - n16, mask=lane0)
                return count_a, count_n

            count_a, count_n = lax.cond(
                found, do_append, lambda c: c, (count_a, count_n))
            return jnp.where(found, j + 1, B), count_a, count_n

        lax.while_loop(lambda c: c[0] < B, pass_body,
                       (jnp.int32(_Q), jnp.int32(len(_A_ROWS)),
                        jnp.int32(len(_N_ROWS))))
        pltpu.sync_copy(outs_v, out_hbm)


def _memory_loop(s, binit, oinit):
    mesh = plsc.VectorSubcoreMesh(core_axis_name="c", subcore_axis_name="s", num_cores=1)
    perm = jnp.asarray(_PERM)
    run = pl.kernel(
        _loop_body,
        out_type=jax.ShapeDtypeStruct((B,), jnp.float32),
        mesh=mesh,
        scratch_types=[
            pltpu.VMEM((B,), jnp.int32),
            pltpu.VMEM((B,), jnp.float32),
            pltpu.VMEM((B,), jnp.float32),
            pltpu.VMEM((B,), jnp.float32),
            pltpu.VMEM((B,), jnp.float32),
            pltpu.SemaphoreType.DMA,
        ],
        compiler_params=pltpu.CompilerParams(needs_layout_passes=False),
    )
    return run(s, perm, binit, oinit)


@jax.jit
def kernel(x, W0, b0):
    binit, oinit = _h_and_boot(x, W0, b0)
    outs_fast, jacc = _fast_sweep(binit, oinit)
    no_appends = jnp.min(jacc) >= B

    def fast(_):
        return outs_fast

    def slow(_):
        return _memory_loop(_full_sim(x, W0, b0), binit, oinit)

    return lax.cond(no_appends, fast, slow, operand=None)


# speedup vs baseline: 1.1416x; 1.1416x over previous
import functools

import numpy as np
import jax
import jax.numpy as jnp
from jax import lax
from jax.experimental import pallas as pl
from jax.experimental.pallas import tpu as pltpu
from jax.experimental.pallas import tpu_sc as plsc

B = 1024
D_IN = 2048
D_H = 512
HALF = B // 2
THR = np.float32(0.2)
L = 16
RB = 256
NRB = B // RB

_PERM = np.random.RandomState(0).permutation(B).astype(np.int32)

_BOOT = []
_mem = {0: [], 1: []}
for _p in range(B):
    if len(_mem[0]) >= 2 and len(_mem[1]) >= 2:
        break
    _idx = int(_PERM[_p])
    _c = 0 if _idx < HALF else 1
    if len(_mem[_c]) >= 2:
        raise AssertionError("bootstrap interrupted; generic prologue needed")
    _mem[_c].append(_idx)
    _BOOT.append((_idx, _c, [list(_mem[0]), list(_mem[1])]))
_Q = len(_BOOT)
_A_ROWS = _mem[0]
_N_ROWS = _mem[1]
_SEED_ROWS = _A_ROWS + _N_ROWS


def _boot_products(sim_rows_of):
    binit_a = jnp.maximum(sim_rows_of(_A_ROWS[0]), sim_rows_of(_A_ROWS[1]))
    binit_n = jnp.maximum(sim_rows_of(_N_ROWS[0]), sim_rows_of(_N_ROWS[1]))
    pos = lax.broadcasted_iota(jnp.int32, (1, B), 1)
    oinit = jnp.zeros((1, B), jnp.float32)
    for _idx, _c, (_ma, _mn) in _BOOT:
        a_s = (jnp.float32(0.5) if not _ma
               else functools.reduce(jnp.maximum,
                                     [sim_rows_of(k)[0, _idx] for k in _ma]))
        n_s = (jnp.float32(0.5) if not _mn
               else functools.reduce(jnp.maximum,
                                     [sim_rows_of(k)[0, _idx] for k in _mn]))
        oinit = jnp.where(pos == _idx, a_s - n_s, oinit)
    return binit_a, binit_n, oinit


_KS = sorted(set(_SEED_ROWS))


def _h_body(x_ref, w_ref, b_ref, binit_ref, oinit_ref, hfull, n2full, hks):
    i = pl.program_id(0)
    h = lax.dot_general(x_ref[...], w_ref[...], (((1,), (1,)), ((), ())),
                        preferred_element_type=jnp.float32)
    h = jnp.maximum(h + b_ref[...], 0.0)
    hfull[pl.ds(i * RB, RB), :] = h
    n2full[pl.ds(i * RB, RB), :] = jnp.sum(h * h, axis=1, keepdims=True)
    for r, k in enumerate(_KS):
        @pl.when(i == k // RB)
        def _stash(r=r, o=k % RB):
            hks[r:r + 1, :] = h[o:o + 1, :]

    @pl.when(i == NRB - 1)
    def _tail():
        hf = hfull[...]
        norms = jnp.sqrt(n2full[...])
        nums = lax.dot_general(hks[...], hf, (((1,), (1,)), ((), ())),
                               preferred_element_type=jnp.float32)
        nks = jnp.concatenate([norms[k:k + 1, :] for k in _KS], axis=0)
        sims = nums / jnp.maximum(nks * norms.T, 1e-8)
        rows = {k: sims[r:r + 1, :] for r, k in enumerate(_KS)}
        binit_a, binit_n, oinit = _boot_products(lambda k: rows[k])
        binit_ref[0:1, :] = binit_a
        binit_ref[1:2, :] = binit_n
        oinit_ref[...] = oinit


def _h_and_boot(x, W0, b0):
    return pl.pallas_call(
        _h_body,
        grid=(NRB,),
        in_specs=[
            pl.BlockSpec((RB, D_IN), lambda i: (i, 0)),
            pl.BlockSpec((D_H, D_IN), lambda i: (0, 0)),
            pl.BlockSpec((1, D_H), lambda i: (0, 0)),
        ],
        out_specs=[
            pl.BlockSpec((2, B), lambda i: (0, 0)),
            pl.BlockSpec((1, B), lambda i: (0, 0)),
        ],
        out_shape=(
            jax.ShapeDtypeStruct((2, B), jnp.float32),
            jax.ShapeDtypeStruct((1, B), jnp.float32),
        ),
        scratch_shapes=[pltpu.VMEM((B, D_H), jnp.float32),
                        pltpu.VMEM((B, 1), jnp.float32),
                        pltpu.VMEM((len(_KS), D_H), jnp.float32)],
    )(x, W0, b0.reshape(1, D_H))


def _s_body(x_ref, w_ref, b_ref, s_ref):
    hf = lax.dot_general(x_ref[...], w_ref[...], (((1,), (1,)), ((), ())),
                         preferred_element_type=jnp.float32)
    hf = jnp.maximum(hf + b_ref[...], 0.0)
    g = lax.dot_general(hf, hf, (((1,), (1,)), ((), ())),
                        preferred_element_type=jnp.float32)
    norms = jnp.sqrt(jnp.sum(hf * hf, axis=1, keepdims=True))
    den = jnp.maximum(norms * norms.T, 1e-8)
    s_ref[...] = g / den


def _full_sim(x, W0, b0):
    return pl.pallas_call(
        _s_body,
        out_shape=jax.ShapeDtypeStruct((B, B), jnp.float32),
    )(x, W0, b0.reshape(1, D_H))


def _sweep_body(perm_hbm, binit_hbm, oinit_hbm, out_hbm, j_hbm,
                perm_v, best_a, best_n, outs_v, j_v, sem):
    wid = lax.axis_index("s") * 2 + lax.axis_index("c")

    @pl.when(wid == 0)
    def _run():
        c1 = pltpu.async_copy(perm_hbm, perm_v, sem)
        c2 = pltpu.async_copy(binit_hbm.at[0], best_a, sem)
        c3 = pltpu.async_copy(binit_hbm.at[1], best_n, sem)
        c4 = pltpu.async_copy(oinit_hbm.at[0], outs_v, sem)
        c1.wait()
        c2.wait()
        c3.wait()
        c4.wait()

        lane = lax.iota(jnp.int32, L)

        @plsc.parallel_loop(0, B // L, 1, unroll=4,
                            carry=jnp.full((L,), B, jnp.int32))
        def acc(k, acc_v):
            pos = k * L + lane
            idxs = perm_v[pl.ds(k * L, L)]
            isa = idxs < HALF
            ba = plsc.load_gather(best_a, [idxs])
            bn = plsc.load_gather(best_n, [idxs])
            cs = jnp.where(isa, ba, bn)
            live = pos >= _Q
            flag = (cs < THR) & live
            plsc.store_scatter(outs_v, [idxs], ba - bn, mask=live)
            return jnp.minimum(acc_v, jnp.where(flag, pos, B))

        j_v[...] = acc
        pltpu.sync_copy(outs_v, out_hbm)
        pltpu.sync_copy(j_v, j_hbm)


def _fast_sweep(binit, oinit):
    mesh = plsc.VectorSubcoreMesh(core_axis_name="c", subcore_axis_name="s", num_cores=1)
    perm = jnp.asarray(_PERM)
    run = pl.kernel(
        _sweep_body,
        out_type=(jax.ShapeDtypeStruct((B,), jnp.float32),
                  jax.ShapeDtypeStruct((L,), jnp.int32)),
        mesh=mesh,
        scratch_types=[
            pltpu.VMEM((B,), jnp.int32),
            pltpu.VMEM((B,), jnp.float32),
            pltpu.VMEM((B,), jnp.float32),
            pltpu.VMEM((B,), jnp.float32),
            pltpu.VMEM((L,), jnp.int32),
            pltpu.SemaphoreType.DMA,
        ],
        compiler_params=pltpu.CompilerParams(needs_layout_passes=False),
    )
    return run(perm, binit, oinit)


def _loop_body(s_hbm, perm_hbm, binit_hbm, oinit_hbm, out_hbm,
               perm_v, best_a, best_n, outs_v, row_v, sem):
    wid = lax.axis_index("s") * 2 + lax.axis_index("c")

    @pl.when(wid == 0)
    def _run():
        c1 = pltpu.async_copy(perm_hbm, perm_v, sem)
        c2 = pltpu.async_copy(binit_hbm.at[0], best_a, sem)
        c3 = pltpu.async_copy(binit_hbm.at[1], best_n, sem)
        c4 = pltpu.async_copy(oinit_hbm.at[0], outs_v, sem)
        c1.wait()
        c2.wait()
        c3.wait()
        c4.wait()

        lane = lax.iota(jnp.int32, L)
        lane0 = lane == 0

        def pass_body(carry):
            s, count_a, count_n = carry

            def chunk(k, acc):
                pos = k * L + lane
                idxs = perm_v[pl.ds(k * L, L)]
                isa = idxs < HALF
                ba = plsc.load_gather(best_a, [idxs])
                bn = plsc.load_gather(best_n, [idxs])
                cs = jnp.where(isa, ba, bn)
                cnt = jnp.where(isa, count_a, count_n)
                live = pos >= s
                flag = ((cnt <= 1) | (cs < THR)) & live
                plsc.store_scatter(outs_v, [idxs], ba - bn, mask=live)
                return jnp.minimum(acc, jnp.where(flag, pos, B))

            acc = lax.fori_loop(0, B // L, chunk,
                                jnp.full((L,), B, jnp.int32))
            j = jnp.min(acc)
            found = j < B

            def do_append(c):
                count_a, count_n = c
                j16 = jnp.full((L,), 0, jnp.int32) + j
                idx16 = plsc.load_gather(perm_v, [j16])
                idx_s = jnp.max(idx16)
                is_a = idx_s < HALF

                pltpu.sync_copy(s_hbm.at[idx_s], row_v)

                @pl.when(is_a)
                def _ua():
                    for k in range(B // L):
                        sl = pl.ds(k * L, L)
                        best_a[sl] = jnp.maximum(best_a[sl], row_v[sl])

                @pl.when(jnp.logical_not(is_a))
                def _un():
                    for k in range(B // L):
                        sl = pl.ds(k * L, L)
                        best_n[sl] = jnp.maximum(best_n[sl], row_v[sl])

                count_a = count_a + jnp.where(is_a, 1, 0)
                count_n = count_n + jnp.where(is_a, 0, 1)
                a16 = plsc.load_gather(best_a, [idx16])
                n16 = plsc.load_gather(best_n, [idx16])
                plsc.store_scatter(outs_v, [idx16], a16 - n16, mask=lane0)
                return count_a, count_n

            count_a, count_n = lax.cond(
                found, do_append, lambda c: c, (count_a, count_n))
            return jnp.where(found, j + 1, B), count_a, count_n

        lax.while_loop(lambda c: c[0] < B, pass_body,
                       (jnp.int32(_Q), jnp.int32(len(_A_ROWS)),
                        jnp.int32(len(_N_ROWS))))
        pltpu.sync_copy(outs_v, out_hbm)


def _memory_loop(s, binit, oinit):
    mesh = plsc.VectorSubcoreMesh(core_axis_name="c", subcore_axis_name="s", num_cores=1)
    perm = jnp.asarray(_PERM)
    run = pl.kernel(
        _loop_body,
        out_type=jax.ShapeDtypeStruct((B,), jnp.float32),
        mesh=mesh,
        scratch_types=[
            pltpu.VMEM((B,), jnp.int32),
            pltpu.VMEM((B,), jnp.float32),
            pltpu.VMEM((B,), jnp.float32),
            pltpu.VMEM((B,), jnp.float32),
            pltpu.VMEM((B,), jnp.float32),
            pltpu.SemaphoreType.DMA,
        ],
        compiler_params=pltpu.CompilerParams(needs_layout_passes=False),
    )
    return run(s, perm, binit, oinit)


@jax.jit
def kernel(x, W0, b0):
    binit, oinit = _h_and_boot(x, W0, b0)
    outs_fast, jacc = _fast_sweep(binit, oinit)
    no_appends = jnp.min(jacc) >= B

    def fast(_):
        return outs_fast

    def slow(_):
        return _memory_loop(_full_sim(x, W0, b0), binit, oinit)

    return lax.cond(no_appends, fast, slow, operand=None)


# speedup vs baseline: 1.1857x; 1.0386x over previous
import functools

import numpy as np
import jax
import jax.numpy as jnp
from jax import lax
from jax.experimental import pallas as pl
from jax.experimental.pallas import tpu as pltpu
from jax.experimental.pallas import tpu_sc as plsc

B = 1024
D_IN = 2048
D_H = 512
HALF = B // 2
THR = np.float32(0.2)
L = 16
RB = 512
NRB = B // RB

_PERM = np.random.RandomState(0).permutation(B).astype(np.int32)

_BOOT = []
_mem = {0: [], 1: []}
for _p in range(B):
    if len(_mem[0]) >= 2 and len(_mem[1]) >= 2:
        break
    _idx = int(_PERM[_p])
    _c = 0 if _idx < HALF else 1
    if len(_mem[_c]) >= 2:
        raise AssertionError("bootstrap interrupted; generic prologue needed")
    _mem[_c].append(_idx)
    _BOOT.append((_idx, _c, [list(_mem[0]), list(_mem[1])]))
_Q = len(_BOOT)
_A_ROWS = _mem[0]
_N_ROWS = _mem[1]
_SEED_ROWS = _A_ROWS + _N_ROWS


def _boot_products(sim_rows_of):
    binit_a = jnp.maximum(sim_rows_of(_A_ROWS[0]), sim_rows_of(_A_ROWS[1]))
    binit_n = jnp.maximum(sim_rows_of(_N_ROWS[0]), sim_rows_of(_N_ROWS[1]))
    pos = lax.broadcasted_iota(jnp.int32, (1, B), 1)
    oinit = jnp.zeros((1, B), jnp.float32)
    for _idx, _c, (_ma, _mn) in _BOOT:
        a_s = (jnp.float32(0.5) if not _ma
               else functools.reduce(jnp.maximum,
                                     [sim_rows_of(k)[0, _idx] for k in _ma]))
        n_s = (jnp.float32(0.5) if not _mn
               else functools.reduce(jnp.maximum,
                                     [sim_rows_of(k)[0, _idx] for k in _mn]))
        oinit = jnp.where(pos == _idx, a_s - n_s, oinit)
    return binit_a, binit_n, oinit


_KS = sorted(set(_SEED_ROWS))


def _h_body(x_ref, w_ref, b_ref, binit_ref, oinit_ref, hfull, n2full, hks):
    i = pl.program_id(0)
    h = lax.dot_general(x_ref[...], w_ref[...], (((1,), (1,)), ((), ())),
                        preferred_element_type=jnp.float32)
    h = jnp.maximum(h + b_ref[...], 0.0)
    hfull[pl.ds(i * RB, RB), :] = h
    n2full[pl.ds(i * RB, RB), :] = jnp.sum(h * h, axis=1, keepdims=True)
    for r, k in enumerate(_KS):
        @pl.when(i == k // RB)
        def _stash(r=r, o=k % RB):
            hks[r:r + 1, :] = h[o:o + 1, :]

    @pl.when(i == NRB - 1)
    def _tail():
        hf = hfull[...]
        norms = jnp.sqrt(n2full[...])
        nums = lax.dot_general(hks[...], hf, (((1,), (1,)), ((), ())),
                               preferred_element_type=jnp.float32)
        nks = jnp.concatenate([norms[k:k + 1, :] for k in _KS], axis=0)
        sims = nums / jnp.maximum(nks * norms.T, 1e-8)
        rows = {k: sims[r:r + 1, :] for r, k in enumerate(_KS)}
        binit_a, binit_n, oinit = _boot_products(lambda k: rows[k])
        binit_ref[0:1, :] = binit_a
        binit_ref[1:2, :] = binit_n
        oinit_ref[...] = oinit


def _h_and_boot(x, W0, b0):
    return pl.pallas_call(
        _h_body,
        grid=(NRB,),
        in_specs=[
            pl.BlockSpec((RB, D_IN), lambda i: (i, 0)),
            pl.BlockSpec((D_H, D_IN), lambda i: (0, 0)),
            pl.BlockSpec((1, D_H), lambda i: (0, 0)),
        ],
        out_specs=[
            pl.BlockSpec((2, B), lambda i: (0, 0)),
            pl.BlockSpec((1, B), lambda i: (0, 0)),
        ],
        out_shape=(
            jax.ShapeDtypeStruct((2, B), jnp.float32),
            jax.ShapeDtypeStruct((1, B), jnp.float32),
        ),
        scratch_shapes=[pltpu.VMEM((B, D_H), jnp.float32),
                        pltpu.VMEM((B, 1), jnp.float32),
                        pltpu.VMEM((len(_KS), D_H), jnp.float32)],
    )(x, W0, b0.reshape(1, D_H))


def _s_body(x_ref, w_ref, b_ref, s_ref):
    hf = lax.dot_general(x_ref[...], w_ref[...], (((1,), (1,)), ((), ())),
                         preferred_element_type=jnp.float32)
    hf = jnp.maximum(hf + b_ref[...], 0.0)
    g = lax.dot_general(hf, hf, (((1,), (1,)), ((), ())),
                        preferred_element_type=jnp.float32)
    norms = jnp.sqrt(jnp.sum(hf * hf, axis=1, keepdims=True))
    den = jnp.maximum(norms * norms.T, 1e-8)
    s_ref[...] = g / den


def _full_sim(x, W0, b0):
    return pl.pallas_call(
        _s_body,
        out_shape=jax.ShapeDtypeStruct((B, B), jnp.float32),
    )(x, W0, b0.reshape(1, D_H))


def _sweep_body(perm_hbm, binit_hbm, oinit_hbm, out_hbm, j_hbm,
                perm_v, best_a, best_n, outs_v, j_v, sem):
    wid = lax.axis_index("s") * 2 + lax.axis_index("c")

    @pl.when(wid == 0)
    def _run():
        c1 = pltpu.async_copy(perm_hbm, perm_v, sem)
        c2 = pltpu.async_copy(binit_hbm.at[0], best_a, sem)
        c3 = pltpu.async_copy(binit_hbm.at[1], best_n, sem)
        c4 = pltpu.async_copy(oinit_hbm.at[0], outs_v, sem)
        c1.wait()
        c2.wait()
        c3.wait()
        c4.wait()

        lane = lax.iota(jnp.int32, L)

        @plsc.parallel_loop(0, B // L, 1, unroll=4,
                            carry=jnp.full((L,), B, jnp.int32))
        def acc(k, acc_v):
            pos = k * L + lane
            idxs = perm_v[pl.ds(k * L, L)]
            isa = idxs < HALF
            ba = plsc.load_gather(best_a, [idxs])
            bn = plsc.load_gather(best_n, [idxs])
            cs = jnp.where(isa, ba, bn)
            live = pos >= _Q
            flag = (cs < THR) & live
            plsc.store_scatter(outs_v, [idxs], ba - bn, mask=live)
            return jnp.minimum(acc_v, jnp.where(flag, pos, B))

        j_v[...] = acc
        pltpu.sync_copy(outs_v, out_hbm)
        pltpu.sync_copy(j_v, j_hbm)


def _fast_sweep(binit, oinit):
    mesh = plsc.VectorSubcoreMesh(core_axis_name="c", subcore_axis_name="s", num_cores=1)
    perm = jnp.asarray(_PERM)
    run = pl.kernel(
        _sweep_body,
        out_type=(jax.ShapeDtypeStruct((B,), jnp.float32),
                  jax.ShapeDtypeStruct((L,), jnp.int32)),
        mesh=mesh,
        scratch_types=[
            pltpu.VMEM((B,), jnp.int32),
            pltpu.VMEM((B,), jnp.float32),
            pltpu.VMEM((B,), jnp.float32),
            pltpu.VMEM((B,), jnp.float32),
            pltpu.VMEM((L,), jnp.int32),
            pltpu.SemaphoreType.DMA,
        ],
        compiler_params=pltpu.CompilerParams(needs_layout_passes=False),
    )
    return run(perm, binit, oinit)


def _loop_body(s_hbm, perm_hbm, binit_hbm, oinit_hbm, out_hbm,
               perm_v, best_a, best_n, outs_v, row_v, sem):
    wid = lax.axis_index("s") * 2 + lax.axis_index("c")

    @pl.when(wid == 0)
    def _run():
        c1 = pltpu.async_copy(perm_hbm, perm_v, sem)
        c2 = pltpu.async_copy(binit_hbm.at[0], best_a, sem)
        c3 = pltpu.async_copy(binit_hbm.at[1], best_n, sem)
        c4 = pltpu.async_copy(oinit_hbm.at[0], outs_v, sem)
        c1.wait()
        c2.wait()
        c3.wait()
        c4.wait()

        lane = lax.iota(jnp.int32, L)
        lane0 = lane == 0

        def pass_body(carry):
            s, count_a, count_n = carry

            def chunk(k, acc):
                pos = k * L + lane
                idxs = perm_v[pl.ds(k * L, L)]
                isa = idxs < HALF
                ba = plsc.load_gather(best_a, [idxs])
                bn = plsc.load_gather(best_n, [idxs])
                cs = jnp.where(isa, ba, bn)
                cnt = jnp.where(isa, count_a, count_n)
                live = pos >= s
                flag = ((cnt <= 1) | (cs < THR)) & live
                plsc.store_scatter(outs_v, [idxs], ba - bn, mask=live)
                return jnp.minimum(acc, jnp.where(flag, pos, B))

            acc = lax.fori_loop(0, B // L, chunk,
                                jnp.full((L,), B, jnp.int32))
            j = jnp.min(acc)
            found = j < B

            def do_append(c):
                count_a, count_n = c
                j16 = jnp.full((L,), 0, jnp.int32) + j
                idx16 = plsc.load_gather(perm_v, [j16])
                idx_s = jnp.max(idx16)
                is_a = idx_s < HALF

                pltpu.sync_copy(s_hbm.at[idx_s], row_v)

                @pl.when(is_a)
                def _ua():
                    for k in range(B // L):
                        sl = pl.ds(k * L, L)
                        best_a[sl] = jnp.maximum(best_a[sl], row_v[sl])

                @pl.when(jnp.logical_not(is_a))
                def _un():
                    for k in range(B // L):
                        sl = pl.ds(k * L, L)
                        best_n[sl] = jnp.maximum(best_n[sl], row_v[sl])

                count_a = count_a + jnp.where(is_a, 1, 0)
                count_n = count_n + jnp.where(is_a, 0, 1)
                a16 = plsc.load_gather(best_a, [idx16])
                n16 = plsc.load_gather(best_n, [idx16])
                plsc.store_scatter(outs_v, [idx16], a16 - n16, mask=lane0)
                return count_a, count_n

            count_a, count_n = lax.cond(
                found, do_append, lambda c: c, (count_a, count_n))
            return jnp.where(found, j + 1, B), count_a, count_n

        lax.while_loop(lambda c: c[0] < B, pass_body,
                       (jnp.int32(_Q), jnp.int32(len(_A_ROWS)),
                        jnp.int32(len(_N_ROWS))))
        pltpu.sync_copy(outs_v, out_hbm)


def _memory_loop(s, binit, oinit):
    mesh = plsc.VectorSubcoreMesh(core_axis_name="c", subcore_axis_name="s", num_cores=1)
    perm = jnp.asarray(_PERM)
    run = pl.kernel(
        _loop_body,
        out_type=jax.ShapeDtypeStruct((B,), jnp.float32),
        mesh=mesh,
        scratch_types=[
            pltpu.VMEM((B,), jnp.int32),
            pltpu.VMEM((B,), jnp.float32),
            pltpu.VMEM((B,), jnp.float32),
            pltpu.VMEM((B,), jnp.float32),
            pltpu.VMEM((B,), jnp.float32),
            pltpu.SemaphoreType.DMA,
        ],
        compiler_params=pltpu.CompilerParams(needs_layout_passes=False),
    )
    return run(s, perm, binit, oinit)


@jax.jit
def kernel(x, W0, b0):
    binit, oinit = _h_and_boot(x, W0, b0)
    outs_fast, jacc = _fast_sweep(binit, oinit)
    no_appends = jnp.min(jacc) >= B

    def fast(_):
        return outs_fast

    def slow(_):
        return _memory_loop(_full_sim(x, W0, b0), binit, oinit)

    return lax.cond(no_appends, fast, slow, operand=None)
